# Initial kernel scaffold; baseline (speedup 1.0000x reference)
import functools

import jax
import jax.numpy as jnp
from jax import lax
from jax.experimental import pallas as pl
from jax.experimental.pallas import tpu as pltpu
from jax.experimental.pallas import tpu_sc as plsc

_LANES = 16
_CH = 128
_K = 8


@functools.lru_cache(maxsize=None)
def _build(num_fields: int, vocab: int, embed_dim: int, batch: int):
    info = plsc.get_sparse_core_info()
    nc, ns = info.num_cores, info.num_subcores
    nw = nc * ns
    total = batch * num_fields
    assert total % nw == 0
    rows_per_w = total // nw
    group = _CH * _K
    assert rows_per_w % group == 0
    num_groups = rows_per_w // group
    mesh = plsc.VectorSubcoreMesh(core_axis_name="c", subcore_axis_name="s")

    @functools.partial(
        pl.kernel,
        mesh=mesh,
        out_type=jax.ShapeDtypeStruct((total, embed_dim), jnp.float32),
        scratch_types=[
            pltpu.VMEM((rows_per_w,), jnp.int32),
            pltpu.VMEM((group, embed_dim), jnp.float32),
            pltpu.SemaphoreType.DMA,
        ],
    )
    def emb_kernel(x_hbm, tab_hbm, out_hbm, idx_v, rows_v, sem):
        wid = lax.axis_index("s") * nc + lax.axis_index("c")
        wbase = wid * rows_per_w
        pltpu.sync_copy(x_hbm.at[pl.ds(wbase, rows_per_w)], idx_v)

        lane = lax.iota(jnp.int32, 16)

        def off_body(i, carry):
            sl = pl.ds(i * _LANES, _LANES)
            pos = wbase + i * _LANES + lane
            idx_v[sl] = idx_v[sl] + (pos % num_fields) * vocab
            return carry

        lax.fori_loop(0, rows_per_w // _LANES, off_body, 0)

        def grp_body(g, carry):
            gbase = g * group
            copies = [
                pltpu.async_copy(
                    tab_hbm.at[idx_v.at[pl.ds(gbase + b * _CH, _CH)]],
                    rows_v.at[pl.ds(b * _CH, _CH)],
                    sem,
                )
                for b in range(_K)
            ]
            for c in copies:
                c.wait()
            pltpu.sync_copy(rows_v, out_hbm.at[pl.ds(wbase + gbase, group)])
            return carry

        lax.fori_loop(0, num_groups, grp_body, 0)

    return emb_kernel


def kernel(x, tables):
    batch, num_fields = x.shape
    nf, vocab, embed_dim = tables.shape
    assert nf == num_fields
    emb = _build(num_fields, vocab, embed_dim, batch)
    flat = emb(x.reshape(-1), tables.reshape(num_fields * vocab, embed_dim))
    return flat.reshape(batch, num_fields * embed_dim)


# speedup vs baseline: 1.1159x; 1.1159x over previous
import functools

import jax
import jax.numpy as jnp
from jax import lax
from jax.experimental import pallas as pl
from jax.experimental.pallas import tpu as pltpu
from jax.experimental.pallas import tpu_sc as plsc

_LANES = 16
_CH = 128
_K = 8


@functools.lru_cache(maxsize=None)
def _build(num_fields: int, vocab: int, embed_dim: int, batch: int):
    info = plsc.get_sparse_core_info()
    nc, ns = info.num_cores, info.num_subcores
    nw = nc * ns
    total = batch * num_fields
    assert total % nw == 0
    rows_per_w = total // nw
    group = _CH * _K
    assert rows_per_w % group == 0
    num_groups = rows_per_w // group
    mesh = plsc.VectorSubcoreMesh(core_axis_name="c", subcore_axis_name="s")

    @functools.partial(
        pl.kernel,
        mesh=mesh,
        compiler_params=pltpu.CompilerParams(use_tc_tiling_on_sc=False),
        out_type=jax.ShapeDtypeStruct((total, embed_dim), jnp.float32),
        scratch_types=[
            pltpu.VMEM((rows_per_w,), jnp.int32),
            pltpu.VMEM((group, embed_dim), jnp.float32),
            pltpu.SemaphoreType.DMA,
        ],
    )
    def emb_kernel(x_hbm, tab_hbm, out_hbm, idx_v, rows_v, sem):
        wid = lax.axis_index("s") * nc + lax.axis_index("c")
        wbase = wid * rows_per_w
        pltpu.sync_copy(x_hbm.at[pl.ds(wbase, rows_per_w)], idx_v)

        lane = lax.iota(jnp.int32, 16)

        def off_body(i, carry):
            sl = pl.ds(i * _LANES, _LANES)
            pos = wbase + i * _LANES + lane
            idx_v[sl] = idx_v[sl] + (pos % num_fields) * vocab
            return carry

        lax.fori_loop(0, rows_per_w // _LANES, off_body, 0)

        def grp_body(g, carry):
            gbase = g * group
            copies = [
                pltpu.async_copy(
                    tab_hbm.at[idx_v.at[pl.ds(gbase + b * _CH, _CH)]],
                    rows_v.at[pl.ds(b * _CH, _CH)],
                    sem,
                )
                for b in range(_K)
            ]
            for c in copies:
                c.wait()
            pltpu.sync_copy(rows_v, out_hbm.at[pl.ds(wbase + gbase, group)])
            return carry

        lax.fori_loop(0, num_groups, grp_body, 0)

    return emb_kernel


def kernel(x, tables):
    batch, num_fields = x.shape
    nf, vocab, embed_dim = tables.shape
    assert nf == num_fields
    emb = _build(num_fields, vocab, embed_dim, batch)
    flat = emb(x.reshape(-1), tables.reshape(num_fields * vocab, embed_dim))
    return flat.reshape(batch, num_fields * embed_dim)


# speedup vs baseline: 1.1195x; 1.0032x over previous
import functools

import jax
import jax.numpy as jnp
from jax import lax
from jax.experimental import pallas as pl
from jax.experimental.pallas import tpu as pltpu
from jax.experimental.pallas import tpu_sc as plsc

_LANES = 16
_CH = 128
_K = 8


@functools.lru_cache(maxsize=None)
def _build(num_fields: int, vocab: int, embed_dim: int, batch: int):
    info = plsc.get_sparse_core_info()
    nc, ns = info.num_cores, info.num_subcores
    nw = nc * ns
    total = batch * num_fields
    assert total % nw == 0
    rows_per_w = total // nw
    group = _CH * _K
    assert rows_per_w % group == 0
    num_groups = rows_per_w // group
    assert num_groups >= 3
    mesh = plsc.VectorSubcoreMesh(core_axis_name="c", subcore_axis_name="s")

    @functools.partial(
        pl.kernel,
        mesh=mesh,
        compiler_params=pltpu.CompilerParams(use_tc_tiling_on_sc=False),
        out_type=jax.ShapeDtypeStruct((total, embed_dim), jnp.float32),
        scratch_types=[
            pltpu.VMEM((rows_per_w,), jnp.int32),
            pltpu.VMEM((2 * group, embed_dim), jnp.float32),
            pltpu.SemaphoreType.DMA,
            pltpu.SemaphoreType.DMA,
        ],
    )
    def emb_kernel(x_hbm, tab_hbm, out_hbm, idx_v, rows_v, gsem, wsem):
        wid = lax.axis_index("s") * nc + lax.axis_index("c")
        wbase = wid * rows_per_w
        pltpu.sync_copy(x_hbm.at[pl.ds(wbase, rows_per_w)], idx_v)

        lane = lax.iota(jnp.int32, 16)

        def off_body(i, carry):
            sl = pl.ds(i * _LANES, _LANES)
            pos = wbase + i * _LANES + lane
            idx_v[sl] = idx_v[sl] + (pos % num_fields) * vocab
            return carry

        lax.fori_loop(0, rows_per_w // _LANES, off_body, 0)

        def gather_desc(g, buf, b):
            return pltpu.make_async_copy(
                tab_hbm.at[idx_v.at[pl.ds(g * group + b * _CH, _CH)]],
                rows_v.at[pl.ds(buf * group + b * _CH, _CH)],
                gsem,
            )

        def fire(g, buf):
            for b in range(_K):
                gather_desc(g, buf, b).start()

        def drain(g, buf):
            for b in range(_K):
                gather_desc(g, buf, b).wait()

        def write_desc(g, buf):
            return pltpu.make_async_copy(
                rows_v.at[pl.ds(buf * group, group)],
                out_hbm.at[pl.ds(wbase + g * group, group)],
                wsem,
            )

        fire(0, 0)
        fire(1, 1)
        drain(0, 0)
        write_desc(0, 0).start()

        def body(g, carry):
            buf = g % 2
            drain(g, buf)
            write_desc(g, buf).start()
            write_desc(g - 1, 1 - buf).wait()
            fire(g + 1, 1 - buf)
            return carry

        lax.fori_loop(1, num_groups - 1, body, 0)

        gl = num_groups - 1
        drain(gl, gl % 2)
        write_desc(gl, gl % 2).start()
        write_desc(gl - 1, (gl - 1) % 2).wait()
        write_desc(gl, gl % 2).wait()

    return emb_kernel


def kernel(x, tables):
    batch, num_fields = x.shape
    nf, vocab, embed_dim = tables.shape
    assert nf == num_fields
    emb = _build(num_fields, vocab, embed_dim, batch)
    flat = emb(x.reshape(-1), tables.reshape(num_fields * vocab, embed_dim))
    return flat.reshape(batch, num_fields * embed_dim)


# speedup vs baseline: 1.7251x; 1.5409x over previous
import functools

import jax
import jax.numpy as jnp
from jax import lax
from jax.experimental import pallas as pl
from jax.experimental.pallas import tpu as pltpu
from jax.experimental.pallas import tpu_sc as plsc

_LANES = 16
_CHUNK = 4096


@functools.lru_cache(maxsize=None)
def _build(num_fields: int, vocab: int, embed_dim: int, batch: int):
    info = plsc.get_sparse_core_info()
    nc, ns = info.num_cores, info.num_subcores
    nw = nc * ns
    assert embed_dim == nw
    assert batch % _CHUNK == 0
    nchunks = batch // _CHUNK
    assert nchunks % 2 == 0
    mesh = plsc.VectorSubcoreMesh(core_axis_name="c", subcore_axis_name="s")

    @functools.partial(
        pl.kernel,
        mesh=mesh,
        compiler_params=pltpu.CompilerParams(use_tc_tiling_on_sc=False,
                                             needs_layout_passes=False),
        out_type=jax.ShapeDtypeStruct((num_fields * embed_dim, batch),
                                      jnp.float32),
        scratch_types=[
            pltpu.VMEM((vocab,), jnp.float32),
            pltpu.VMEM((2, _CHUNK), jnp.int32),
            pltpu.VMEM((2, _CHUNK), jnp.float32),
            pltpu.SemaphoreType.DMA,
            pltpu.SemaphoreType.DMA,
        ],
    )
    def emb_kernel(xt_hbm, tabt_hbm, out_hbm, row_v, xbuf, obuf, xsem, wsem):
        e = lax.axis_index("s") * nc + lax.axis_index("c")

        def field_body(f, carry):
            pltpu.sync_copy(tabt_hbm.at[f, e], row_v)
            orow = f * embed_dim + e
            pltpu.async_copy(xt_hbm.at[f, pl.ds(0, _CHUNK)], xbuf.at[0],
                             xsem).wait()

            for c in range(nchunks):
                p = c % 2
                if c + 1 < nchunks:
                    nxt = pltpu.async_copy(
                        xt_hbm.at[f, pl.ds((c + 1) * _CHUNK, _CHUNK)],
                        xbuf.at[1 - p], xsem)
                if c >= 2:
                    pltpu.make_async_copy(
                        obuf.at[p],
                        out_hbm.at[orow, pl.ds((c - 2) * _CHUNK, _CHUNK)],
                        wsem,
                    ).wait()

                def gat(i, carry2):
                    sl = pl.ds(i * _LANES, _LANES)
                    obuf[p, sl] = plsc.load_gather(row_v, [xbuf[p, sl]])
                    return carry2

                lax.fori_loop(0, _CHUNK // _LANES, gat, 0)
                pltpu.make_async_copy(
                    obuf.at[p],
                    out_hbm.at[orow, pl.ds(c * _CHUNK, _CHUNK)],
                    wsem,
                ).start()
                if c + 1 < nchunks:
                    nxt.wait()

            for p in (0, 1):
                pltpu.make_async_copy(
                    obuf.at[p],
                    out_hbm.at[orow, pl.ds((nchunks - 2 + p) * _CHUNK,
                                           _CHUNK)],
                    wsem,
                ).wait()
            return carry

        lax.fori_loop(0, num_fields, field_body, 0)

    return emb_kernel


def kernel(x, tables):
    batch, num_fields = x.shape
    nf, vocab, embed_dim = tables.shape
    assert nf == num_fields
    emb = _build(num_fields, vocab, embed_dim, batch)
    out_t = emb(x.T, tables.transpose(0, 2, 1))
    return out_t.T.reshape(batch, num_fields * embed_dim)


# speedup vs baseline: 1.8274x; 1.0593x over previous
import functools

import jax
import jax.numpy as jnp
from jax import lax
from jax.experimental import pallas as pl
from jax.experimental.pallas import tpu as pltpu
from jax.experimental.pallas import tpu_sc as plsc

_LANES = 16
_CHUNK = 4096
_UNROLL = 16


@functools.lru_cache(maxsize=None)
def _build(num_fields: int, vocab: int, embed_dim: int, batch: int):
    info = plsc.get_sparse_core_info()
    nc, ns = info.num_cores, info.num_subcores
    nw = nc * ns
    assert embed_dim == nw
    assert batch % _CHUNK == 0
    nchunks = batch // _CHUNK
    assert nchunks % 2 == 0
    mesh = plsc.VectorSubcoreMesh(core_axis_name="c", subcore_axis_name="s")

    @functools.partial(
        pl.kernel,
        mesh=mesh,
        compiler_params=pltpu.CompilerParams(use_tc_tiling_on_sc=False,
                                             needs_layout_passes=False),
        out_type=jax.ShapeDtypeStruct((num_fields * embed_dim, batch),
                                      jnp.float32),
        scratch_types=[
            pltpu.VMEM((vocab,), jnp.float32),
            pltpu.VMEM((2, _CHUNK), jnp.int32),
            pltpu.VMEM((2, _CHUNK), jnp.float32),
            pltpu.SemaphoreType.DMA,
            pltpu.SemaphoreType.DMA,
        ],
    )
    def emb_kernel(xt_hbm, tabt_hbm, out_hbm, row_v, xbuf, obuf, xsem, wsem):
        e = lax.axis_index("s") * nc + lax.axis_index("c")

        def field_body(f, carry):
            pltpu.sync_copy(tabt_hbm.at[f, e], row_v)
            orow = f * embed_dim + e
            pltpu.async_copy(xt_hbm.at[f, pl.ds(0, _CHUNK)], xbuf.at[0],
                             xsem).wait()

            for c in range(nchunks):
                p = c % 2
                if c + 1 < nchunks:
                    nxt = pltpu.async_copy(
                        xt_hbm.at[f, pl.ds((c + 1) * _CHUNK, _CHUNK)],
                        xbuf.at[1 - p], xsem)
                if c >= 2:
                    pltpu.make_async_copy(
                        obuf.at[p],
                        out_hbm.at[orow, pl.ds((c - 2) * _CHUNK, _CHUNK)],
                        wsem,
                    ).wait()

                def gat(i, carry2):
                    for u in range(_UNROLL):
                        sl = pl.ds((i * _UNROLL + u) * _LANES, _LANES)
                        obuf[p, sl] = plsc.load_gather(row_v, [xbuf[p, sl]])
                    return carry2

                lax.fori_loop(0, _CHUNK // (_LANES * _UNROLL), gat, 0)
                pltpu.make_async_copy(
                    obuf.at[p],
                    out_hbm.at[orow, pl.ds(c * _CHUNK, _CHUNK)],
                    wsem,
                ).start()
                if c + 1 < nchunks:
                    nxt.wait()

            for p in (0, 1):
                pltpu.make_async_copy(
                    obuf.at[p],
                    out_hbm.at[orow, pl.ds((nchunks - 2 + p) * _CHUNK,
                                           _CHUNK)],
                    wsem,
                ).wait()
            return carry

        lax.fori_loop(0, num_fields, field_body, 0)

    return emb_kernel


def kernel(x, tables):
    batch, num_fields = x.shape
    nf, vocab, embed_dim = tables.shape
    assert nf == num_fields
    emb = _build(num_fields, vocab, embed_dim, batch)
    out_t = emb(x.T, tables.transpose(0, 2, 1))
    return out_t.T.reshape(batch, num_fields * embed_dim)


# speedup vs baseline: 1.8315x; 1.0023x over previous
import functools

import jax
import jax.numpy as jnp
from jax import lax
from jax.experimental import pallas as pl
from jax.experimental.pallas import tpu as pltpu
from jax.experimental.pallas import tpu_sc as plsc

_LANES = 16
_CHUNK = 4096
_UNROLL = 16


@functools.lru_cache(maxsize=None)
def _build(num_fields: int, vocab: int, embed_dim: int, batch: int):
    info = plsc.get_sparse_core_info()
    nc, ns = info.num_cores, info.num_subcores
    nw = nc * ns
    assert embed_dim == nw
    assert batch % _CHUNK == 0
    nchunks = batch // _CHUNK
    assert nchunks % 2 == 0
    mesh = plsc.VectorSubcoreMesh(core_axis_name="c", subcore_axis_name="s")

    @functools.partial(
        pl.kernel,
        mesh=mesh,
        compiler_params=pltpu.CompilerParams(use_tc_tiling_on_sc=False,
                                             needs_layout_passes=False),
        out_type=jax.ShapeDtypeStruct((num_fields * embed_dim, batch),
                                      jnp.float32),
        scratch_types=[
            pltpu.VMEM((vocab,), jnp.float32),
            pltpu.VMEM((2, _CHUNK), jnp.int32),
            pltpu.VMEM((2, _CHUNK), jnp.float32),
            pltpu.SemaphoreType.DMA,
            pltpu.SemaphoreType.DMA,
            pltpu.SemaphoreType.DMA,
        ],
    )
    def emb_kernel(xt_hbm, tabt_hbm, out_hbm, row_v, xbuf, obuf, xsem, wsem,
                   rsem):
        e = lax.axis_index("s") * nc + lax.axis_index("c")
        nq = 4
        qlen = vocab // nq

        def field_body(f, carry):
            qcopies = [
                pltpu.async_copy(
                    tabt_hbm.at[f, e, pl.ds(q * qlen, qlen)],
                    row_v.at[pl.ds(q * qlen, qlen)],
                    rsem,
                )
                for q in range(nq)
            ]
            for qc in qcopies:
                qc.wait()
            orow = f * embed_dim + e
            pltpu.async_copy(xt_hbm.at[f, pl.ds(0, _CHUNK)], xbuf.at[0],
                             xsem).wait()

            for c in range(nchunks):
                p = c % 2
                if c + 1 < nchunks:
                    nxt = pltpu.async_copy(
                        xt_hbm.at[f, pl.ds((c + 1) * _CHUNK, _CHUNK)],
                        xbuf.at[1 - p], xsem)
                if c >= 2:
                    pltpu.make_async_copy(
                        obuf.at[p],
                        out_hbm.at[orow, pl.ds((c - 2) * _CHUNK, _CHUNK)],
                        wsem,
                    ).wait()

                def gat(i, carry2):
                    for u in range(_UNROLL):
                        sl = pl.ds((i * _UNROLL + u) * _LANES, _LANES)
                        obuf[p, sl] = plsc.load_gather(row_v, [xbuf[p, sl]])
                    return carry2

                lax.fori_loop(0, _CHUNK // (_LANES * _UNROLL), gat, 0)
                pltpu.make_async_copy(
                    obuf.at[p],
                    out_hbm.at[orow, pl.ds(c * _CHUNK, _CHUNK)],
                    wsem,
                ).start()
                if c + 1 < nchunks:
                    nxt.wait()

            for p in (0, 1):
                pltpu.make_async_copy(
                    obuf.at[p],
                    out_hbm.at[orow, pl.ds((nchunks - 2 + p) * _CHUNK,
                                           _CHUNK)],
                    wsem,
                ).wait()
            return carry

        lax.fori_loop(0, num_fields, field_body, 0)

    return emb_kernel


def kernel(x, tables):
    batch, num_fields = x.shape
    nf, vocab, embed_dim = tables.shape
    assert nf == num_fields
    emb = _build(num_fields, vocab, embed_dim, batch)
    out_t = emb(x.T, tables.transpose(0, 2, 1))
    return out_t.T.reshape(batch, num_fields * embed_dim)
